# baseline (speedup 1.0000x reference)
import functools

import jax
import jax.numpy as jnp
from jax import lax
from jax.experimental import pallas as pl
from jax.experimental.pallas import tpu as pltpu
from jax.experimental.pallas import tpu_sc as plsc

N = 10000
NSC = 10016
DUMMY = 10000
E_RAW = 160000
E_TOT = E_RAW + N
E_PAD = 172032
ROW_BLK = 2504

L1 = (128, 200, 208)
L2 = (208, 100, 112)
L3 = (112, 2, 16)


def _augment(h, beta, mpad):
    sq = jnp.sum(h * h, axis=1, keepdims=True)
    r = 1.0 / (jnp.sqrt(sq) + 1e-8)
    col = jax.lax.broadcasted_iota(jnp.int32, h.shape, 1)
    h = jnp.where(col == mpad - 3, r, h)
    h = jnp.where(col == mpad - 2, beta * r, h)
    h = jnp.where(col == mpad - 1, 1.0, h)
    return h


def _mm_first_body(x_ref, w_ref, beta_ref, out_ref, *, mpad):
    h = jnp.dot(x_ref[...], w_ref[...], preferred_element_type=jnp.float32)
    out_ref[...] = _augment(h, beta_ref[0, 0], mpad)


def _combine(nm_ref, summed):
    return (nm_ref[0] + nm_ref[1]) if summed else nm_ref[0]


def _mm_mid_body(nm_ref, w_ref, beta_ref, out_ref, *, mpad, summed):
    nm = _combine(nm_ref, summed)
    denom = nm[:, -1:]
    hin = jnp.maximum(nm / (denom + 1e-16), 0.0)
    h = jnp.dot(hin, w_ref[...], preferred_element_type=jnp.float32)
    out_ref[...] = _augment(h, beta_ref[0, 0], mpad)


def _final_body(nm_ref, out_ref, *, summed):
    nm = _combine(nm_ref, summed)
    denom = nm[:, -1:]
    out_ref[...] = jnp.maximum(nm / (denom + 1e-16), 0.0)


def _part_spec(a, summed):
    if summed:
        return pl.BlockSpec((2, ROW_BLK, a.shape[2]), lambda i: (0, i, 0))
    return pl.BlockSpec((1, ROW_BLK, a.shape[2]), lambda i: (i // 2, i % 2, 0))


def _mm_call(body, a, w, beta, mpad, summed=False):
    grid = NSC // ROW_BLK
    if a.ndim == 3:
        a_spec = _part_spec(a, summed)
        body = functools.partial(body, summed=summed)
    else:
        a_spec = pl.BlockSpec((ROW_BLK, a.shape[1]), lambda i: (i, 0))
    k = a.shape[-1]
    return pl.pallas_call(
        functools.partial(body, mpad=mpad),
        grid=(grid,),
        in_specs=[
            a_spec,
            pl.BlockSpec((k, mpad), lambda i: (0, 0)),
            pl.BlockSpec(memory_space=pltpu.SMEM),
        ],
        out_specs=pl.BlockSpec((ROW_BLK, mpad), lambda i: (i, 0)),
        out_shape=jax.ShapeDtypeStruct((NSC, mpad), jnp.float32),
    )(a, w, beta)


def _final_call(nm, summed=False):
    mpad = nm.shape[-1]
    grid = NSC // ROW_BLK
    return pl.pallas_call(
        functools.partial(_final_body, summed=summed),
        grid=(grid,),
        in_specs=[_part_spec(nm, summed)],
        out_specs=pl.BlockSpec((ROW_BLK, mpad), lambda i: (i, 0)),
        out_shape=jax.ShapeDtypeStruct((NSC, mpad), jnp.float32),
    )(nm)


CHUNK = 64
PER_TILE = E_PAD // 16
N_CHUNKS = PER_TILE // CHUNK
NHALF = NSC // 2
ACC_ROWS = 5120
TRASH = ACC_ROWS - 1
ACC_FULL = 10240


@functools.lru_cache
def _make_edge_kernel(mpad, mt, esplit):
    nch = mpad // 16
    nfull = mt // 16
    rem = mt % 16
    acc_rows = ACC_FULL if esplit else ACC_ROWS
    nct = (N_CHUNKS // 2) if esplit else N_CHUNKS
    super_ = 28 if esplit else 24
    zrows = acc_rows // 16
    mesh = plsc.VectorSubcoreMesh(core_axis_name="c", subcore_axis_name="s")

    @functools.partial(
        pl.kernel,
        out_type=jax.ShapeDtypeStruct((2, acc_rows, mpad), jnp.float32),
        mesh=mesh,
        compiler_params=pltpu.CompilerParams(use_tc_tiling_on_sc=False),
        scratch_types=[
            pltpu.VMEM((super_ * 2 * CHUNK,), jnp.int32),
            pltpu.VMEM((CHUNK,), jnp.int32),
            pltpu.VMEM((2 * CHUNK, mpad), jnp.float32),
            pltpu.VMEM((2 * CHUNK, mpad), jnp.float32),
            pltpu.VMEM_SHARED((acc_rows, mpad), jnp.float32),
            pltpu.SemaphoreType.DMA,
            pltpu.SemaphoreType.DMA,
        ],
    )
    def edge_kernel(h_hbm, eidx_hbm, out_hbm,
                    idxbuf, didx, bufa, bufb, acc, sem1, sem2):
        cid = lax.axis_index("c")
        sid = lax.axis_index("s")

        def zrow(i, _):
            for c in range(nch):
                bufa[i, pl.ds(c * 16, 16)] = jnp.zeros((16,), jnp.float32)
            return 0
        lax.fori_loop(0, 2 * CHUNK, zrow, 0)
        zbase = pl.multiple_of(sid * zrows, 8)
        for t in range(zrows // (2 * CHUNK)):
            pltpu.sync_copy(bufa, acc.at[pl.ds(zbase + t * 2 * CHUNK,
                                               2 * CHUNK)])
        ztail = zrows % (2 * CHUNK)
        if ztail:
            pltpu.sync_copy(bufa.at[pl.ds(0, ztail)],
                            acc.at[pl.ds(zbase + zrows - ztail, ztail)])
        plsc.subcore_barrier()

        lane = lax.iota(jnp.int32, 16)
        if esplit:
            base = (cid * 16 + sid) * nct * 2 * CHUNK
        else:
            base = sid * nct * 2 * CHUNK
        nbase = cid * NHALF

        def process(buf, jb):
            for q in range(CHUNK // 16):
                sl = pl.ds(q * 16, 16)
                dv = idxbuf[pl.ds(jb + CHUNK + q * 16, 16)]
                if esplit:
                    didx[sl] = dv
                else:
                    dl = dv - nbase
                    ok = (dl >= 0) & (dl < NHALF)
                    trash = NHALF + jnp.bitwise_and(dv, 63)
                    didx[sl] = jnp.where(ok, dl, trash)

            for g in range(CHUNK // 16):
                def edge(i, _):
                    e = g * 16 + i
                    dvec = didx[pl.ds(g * 16, 16)]

                    def work():
                        ts = buf[e, pl.ds((nch - 1) * 16, 16)]
                        td = buf[CHUNK + e, pl.ds((nch - 1) * 16, 16)]
                        acc16 = jnp.where(lane < rem, ts * td, 0.0)
                        for c in range(nfull):
                            acc16 += (buf[e, pl.ds(c * 16, 16)]
                                      * buf[CHUNK + e, pl.ds(c * 16, 16)])
                        for k in (8, 4, 2, 1):
                            acc16 = acc16 + acc16.at[
                                jnp.bitwise_xor(lane, k)].get(
                                    mode="promise_in_bounds")
                        exb = jnp.exp(ts[14] * td[13] * acc16)
                        for c in range(nch):
                            sl = pl.ds(c * 16, 16)
                            buf[e, sl] = buf[e, sl] * exb

                    if esplit:
                        work()
                    else:
                        db = dvec.at[jnp.broadcast_to(i, (16,)).astype(
                            jnp.int32)].get(mode="promise_in_bounds")
                        work()
                    return 0

                lax.fori_loop(0, 16, edge, 0)

            pltpu.sync_copy(buf.at[pl.ds(0, CHUNK)], acc.at[didx], add=True)

        def block(t, _):
            boff = pl.multiple_of(base + t * super_ * 2 * CHUNK, 8)
            pltpu.sync_copy(eidx_hbm.at[pl.ds(boff, super_ * 2 * CHUNK)],
                            idxbuf)

            def pair(k, _):
                ja = pl.multiple_of(k * 4 * CHUNK, 8)
                jb = ja + 2 * CHUNK
                cpa = pltpu.async_copy(
                    h_hbm.at[idxbuf.at[pl.ds(ja, 2 * CHUNK)]], bufa, sem1)
                cpb = pltpu.async_copy(
                    h_hbm.at[idxbuf.at[pl.ds(jb, 2 * CHUNK)]], bufb, sem2)
                cpa.wait()
                process(bufa, ja)
                cpb.wait()
                process(bufb, jb)
                return 0

            lax.fori_loop(0, super_ // 2, pair, 0)
            return 0

        lax.fori_loop(0, nct // super_, block, 0)
        plsc.subcore_barrier()

        for t in range(zrows // (2 * CHUNK)):
            sl = pl.ds(zbase + t * 2 * CHUNK, 2 * CHUNK)
            pltpu.sync_copy(acc.at[sl], bufa)
            pltpu.sync_copy(bufa, out_hbm.at[cid, sl])
        tail = zrows % (2 * CHUNK)
        if tail:
            tsl = pl.ds(zbase + zrows - tail, tail)
            pltpu.sync_copy(acc.at[tsl], bufa.at[pl.ds(0, tail)])
            pltpu.sync_copy(bufa.at[pl.ds(0, tail)], out_hbm.at[cid, tsl])

    return edge_kernel


def _edge_phase(h, eidx, mt, esplit=False):
    mpad = h.shape[1]
    return _make_edge_kernel(mpad, mt, esplit)(h, eidx)


def kernel(x, edge_index, W1, beta1, W2, beta2, W3, beta3):
    xp = jnp.ones((NSC, L1[0]), jnp.float32).at[:N].set(x)
    loop = jnp.arange(N, dtype=jnp.int32)
    pad = DUMMY + jnp.arange(E_PAD - E_TOT, dtype=jnp.int32) % (NSC - DUMMY)
    src = jnp.concatenate([edge_index[0].astype(jnp.int32), loop, pad])
    dst = jnp.concatenate([edge_index[1].astype(jnp.int32), loop, pad])
    eidx = jnp.concatenate(
        [src.reshape(-1, CHUNK), dst.reshape(-1, CHUNK)], axis=1).reshape(-1)

    def wpad(w, spec):
        k, mt, mp = spec
        return jnp.zeros((k, mp), jnp.float32).at[: w.shape[0], :mt].set(w)

    w1p, w2p, w3p = wpad(W1, L1), wpad(W2, L2), wpad(W3, L3)
    b = lambda s: jnp.reshape(s, (1, 1)).astype(jnp.float32)

    h1 = _mm_call(_mm_first_body, xp, w1p, b(beta1), L1[2])
    nm1 = _edge_phase(h1, eidx, L1[1])
    h2 = _mm_call(_mm_mid_body, nm1, w2p, b(beta2), L2[2])
    nm2 = _edge_phase(h2, eidx, L2[1], esplit=True)
    h3 = _mm_call(_mm_mid_body, nm2, w3p, b(beta3), L3[2], summed=True)
    nm3 = _edge_phase(h3, eidx, L3[1], esplit=True)
    out = _final_call(nm3, summed=True)
    return out[:N, : L3[1]]


# speedup vs baseline: 1.3786x; 1.3786x over previous
import functools

import jax
import jax.numpy as jnp
from jax import lax
from jax.experimental import pallas as pl
from jax.experimental.pallas import tpu as pltpu
from jax.experimental.pallas import tpu_sc as plsc

N = 10000
NSC = 10016
DUMMY = 10000
E_RAW = 160000
E_TOT = E_RAW + N
E_PAD = 172032
ROW_BLK = 2504

L1 = (128, 200, 208)
L2 = (208, 100, 112)
L3 = (112, 2, 16)


def _augment(h, beta, mpad):
    sq = jnp.sum(h * h, axis=1, keepdims=True)
    r = 1.0 / (jnp.sqrt(sq) + 1e-8)
    col = jax.lax.broadcasted_iota(jnp.int32, h.shape, 1)
    h = jnp.where(col == mpad - 3, r, h)
    h = jnp.where(col == mpad - 2, beta * r, h)
    h = jnp.where(col == mpad - 1, 1.0, h)
    return h


def _mm_first_body(x_ref, w_ref, beta_ref, out_ref, *, mpad):
    h = jnp.dot(x_ref[...], w_ref[...], preferred_element_type=jnp.float32)
    out_ref[...] = _augment(h, beta_ref[0, 0], mpad)


def _combine(nm_ref, summed):
    return (nm_ref[0] + nm_ref[1]) if summed else nm_ref[0]


def _mm_mid_body(nm_ref, w_ref, beta_ref, out_ref, *, mpad, summed):
    nm = _combine(nm_ref, summed)
    denom = nm[:, -1:]
    hin = jnp.maximum(nm / (denom + 1e-16), 0.0)
    h = jnp.dot(hin, w_ref[...], preferred_element_type=jnp.float32)
    out_ref[...] = _augment(h, beta_ref[0, 0], mpad)


def _final_body(nm_ref, out_ref, *, summed):
    nm = _combine(nm_ref, summed)
    denom = nm[:, -1:]
    out_ref[...] = jnp.maximum(nm / (denom + 1e-16), 0.0)


def _part_spec(a, summed):
    if summed:
        return pl.BlockSpec((2, ROW_BLK, a.shape[2]), lambda i: (0, i, 0))
    return pl.BlockSpec((1, ROW_BLK, a.shape[2]), lambda i: (i // 2, i % 2, 0))


def _mm_call(body, a, w, beta, mpad, summed=False):
    grid = NSC // ROW_BLK
    if a.ndim == 3:
        a_spec = _part_spec(a, summed)
        body = functools.partial(body, summed=summed)
    else:
        a_spec = pl.BlockSpec((ROW_BLK, a.shape[1]), lambda i: (i, 0))
    k = a.shape[-1]
    return pl.pallas_call(
        functools.partial(body, mpad=mpad),
        grid=(grid,),
        in_specs=[
            a_spec,
            pl.BlockSpec((k, mpad), lambda i: (0, 0)),
            pl.BlockSpec(memory_space=pltpu.SMEM),
        ],
        out_specs=pl.BlockSpec((ROW_BLK, mpad), lambda i: (i, 0)),
        out_shape=jax.ShapeDtypeStruct((NSC, mpad), jnp.float32),
    )(a, w, beta)


def _final_call(nm, summed=False):
    mpad = nm.shape[-1]
    grid = NSC // ROW_BLK
    return pl.pallas_call(
        functools.partial(_final_body, summed=summed),
        grid=(grid,),
        in_specs=[_part_spec(nm, summed)],
        out_specs=pl.BlockSpec((ROW_BLK, mpad), lambda i: (i, 0)),
        out_shape=jax.ShapeDtypeStruct((NSC, mpad), jnp.float32),
    )(nm)


CHUNK = 64
PER_TILE = E_PAD // 16
N_CHUNKS = PER_TILE // CHUNK
NHALF = NSC // 2
ACC_ROWS = 5120
TRASH = ACC_ROWS - 1
ACC_FULL = 10240


@functools.lru_cache
def _make_edge_kernel(mpad, mt, esplit):
    nch = mpad // 16
    nfull = mt // 16
    rem = mt % 16
    acc_rows = ACC_FULL if esplit else ACC_ROWS
    nct = (N_CHUNKS // 2) if esplit else N_CHUNKS
    super_ = 42
    zrows = acc_rows // 16
    mesh = plsc.VectorSubcoreMesh(core_axis_name="c", subcore_axis_name="s")

    @functools.partial(
        pl.kernel,
        out_type=jax.ShapeDtypeStruct((2, acc_rows, mpad), jnp.float32),
        mesh=mesh,
        compiler_params=pltpu.CompilerParams(use_tc_tiling_on_sc=False),
        scratch_types=[
            pltpu.VMEM((super_ * 2 * CHUNK,), jnp.int32),
            pltpu.VMEM((CHUNK,), jnp.int32),
            pltpu.VMEM((CHUNK,), jnp.int32),
            pltpu.VMEM((2 * CHUNK, mpad), jnp.float32),
            pltpu.VMEM((2 * CHUNK, mpad), jnp.float32),
            pltpu.VMEM_SHARED((acc_rows, mpad), jnp.float32),
            pltpu.SemaphoreType.DMA,
            pltpu.SemaphoreType.DMA,
            pltpu.SemaphoreType.DMA,
        ],
    )
    def edge_kernel(h_hbm, eidx_hbm, out_hbm,
                    idxbuf, didxa, didxb, bufa, bufb, acc, sem1, sem2, sem3):
        cid = lax.axis_index("c")
        sid = lax.axis_index("s")

        def zrow(i, _):
            for c in range(nch):
                bufa[i, pl.ds(c * 16, 16)] = jnp.zeros((16,), jnp.float32)
            return 0
        lax.fori_loop(0, 2 * CHUNK, zrow, 0)
        zbase = pl.multiple_of(sid * zrows, 8)
        for t in range(zrows // (2 * CHUNK)):
            pltpu.sync_copy(bufa, acc.at[pl.ds(zbase + t * 2 * CHUNK,
                                               2 * CHUNK)])
        ztail = zrows % (2 * CHUNK)
        if ztail:
            pltpu.sync_copy(bufa.at[pl.ds(0, ztail)],
                            acc.at[pl.ds(zbase + zrows - ztail, ztail)])
        plsc.subcore_barrier()

        lane = lax.iota(jnp.int32, 16)
        if esplit:
            base = (cid * 16 + sid) * nct * 2 * CHUNK
        else:
            base = sid * nct * 2 * CHUNK
        nbase = cid * NHALF

        def process(buf, jb, didx):
            for q in range(CHUNK // 16):
                sl = pl.ds(q * 16, 16)
                dv = idxbuf[pl.ds(jb + CHUNK + q * 16, 16)]
                if esplit:
                    didx[sl] = dv
                else:
                    dl = dv - nbase
                    ok = (dl >= 0) & (dl < NHALF)
                    trash = NHALF + jnp.bitwise_and(dv, 63)
                    didx[sl] = jnp.where(ok, dl, trash)

            for g in range(CHUNK // 16):
                def edot(i, lgvec):
                    e = g * 16 + i
                    ts = buf[e, pl.ds((nch - 1) * 16, 16)]
                    td = buf[CHUNK + e, pl.ds((nch - 1) * 16, 16)]
                    acc16 = jnp.where(lane < rem, ts * td, 0.0)
                    for c in range(nfull):
                        acc16 += (buf[e, pl.ds(c * 16, 16)]
                                  * buf[CHUNK + e, pl.ds(c * 16, 16)])
                    for k in (8, 4, 2, 1):
                        acc16 = acc16 + acc16.at[
                            jnp.bitwise_xor(lane, k)].get(
                                mode="promise_in_bounds")
                    logit = ts[14] * td[13] * acc16
                    return jnp.where(lane == i, logit, lgvec)
                lgvec = lax.fori_loop(0, 16, edot,
                                      jnp.zeros((16,), jnp.float32))
                exvec = jnp.exp(lgvec)

                def escale(i, _):
                    e = g * 16 + i
                    eidx = jnp.broadcast_to(i, (16,)).astype(jnp.int32)
                    exb = exvec.at[eidx].get(mode="promise_in_bounds")
                    for c in range(nch):
                        sl = pl.ds(c * 16, 16)
                        buf[e, sl] = buf[e, sl] * exb
                    return 0
                lax.fori_loop(0, 16, escale, 0)

        def block(t, _):
            boff = pl.multiple_of(base + t * super_ * 2 * CHUNK, 8)
            pltpu.sync_copy(eidx_hbm.at[pl.ds(boff, super_ * 2 * CHUNK)],
                            idxbuf)

            def pair(k, _):
                ja = pl.multiple_of(k * 4 * CHUNK, 8)
                jb = ja + 2 * CHUNK
                cpa = pltpu.async_copy(
                    h_hbm.at[idxbuf.at[pl.ds(ja, 2 * CHUNK)]], bufa, sem1)
                cpb = pltpu.async_copy(
                    h_hbm.at[idxbuf.at[pl.ds(jb, 2 * CHUNK)]], bufb, sem2)
                cpa.wait()
                process(bufa, ja, didxa)
                sca = pltpu.async_copy(bufa.at[pl.ds(0, CHUNK)],
                                       acc.at[didxa], sem3, add=True)
                cpb.wait()
                process(bufb, jb, didxb)
                sca.wait()
                pltpu.sync_copy(bufb.at[pl.ds(0, CHUNK)], acc.at[didxb],
                                add=True)
                return 0

            lax.fori_loop(0, super_ // 2, pair, 0)
            return 0

        lax.fori_loop(0, nct // super_, block, 0)
        plsc.subcore_barrier()

        for t in range(zrows // (2 * CHUNK)):
            sl = pl.ds(zbase + t * 2 * CHUNK, 2 * CHUNK)
            pltpu.sync_copy(acc.at[sl], bufa)
            pltpu.sync_copy(bufa, out_hbm.at[cid, sl])
        tail = zrows % (2 * CHUNK)
        if tail:
            tsl = pl.ds(zbase + zrows - tail, tail)
            pltpu.sync_copy(acc.at[tsl], bufa.at[pl.ds(0, tail)])
            pltpu.sync_copy(bufa.at[pl.ds(0, tail)], out_hbm.at[cid, tsl])

    return edge_kernel


def _edge_phase(h, eidx, mt, esplit=False):
    mpad = h.shape[1]
    return _make_edge_kernel(mpad, mt, esplit)(h, eidx)


def kernel(x, edge_index, W1, beta1, W2, beta2, W3, beta3):
    xp = jnp.ones((NSC, L1[0]), jnp.float32).at[:N].set(x)
    loop = jnp.arange(N, dtype=jnp.int32)
    pad = DUMMY + jnp.arange(E_PAD - E_TOT, dtype=jnp.int32) % (NSC - DUMMY)
    src = jnp.concatenate([edge_index[0].astype(jnp.int32), loop, pad])
    dst = jnp.concatenate([edge_index[1].astype(jnp.int32), loop, pad])
    eidx = jnp.concatenate(
        [src.reshape(-1, CHUNK), dst.reshape(-1, CHUNK)], axis=1).reshape(-1)

    def wpad(w, spec):
        k, mt, mp = spec
        return jnp.zeros((k, mp), jnp.float32).at[: w.shape[0], :mt].set(w)

    w1p, w2p, w3p = wpad(W1, L1), wpad(W2, L2), wpad(W3, L3)
    b = lambda s: jnp.reshape(s, (1, 1)).astype(jnp.float32)

    h1 = _mm_call(_mm_first_body, xp, w1p, b(beta1), L1[2])
    nm1 = _edge_phase(h1, eidx, L1[1])
    h2 = _mm_call(_mm_mid_body, nm1, w2p, b(beta2), L2[2])
    nm2 = _edge_phase(h2, eidx, L2[1], esplit=True)
    h3 = _mm_call(_mm_mid_body, nm2, w3p, b(beta3), L3[2], summed=True)
    nm3 = _edge_phase(h3, eidx, L3[1], esplit=True)
    out = _final_call(nm3, summed=True)
    return out[:N, : L3[1]]
